# Initial kernel scaffold; baseline (speedup 1.0000x reference)
import functools

import jax
import jax.numpy as jnp
from jax import lax
from jax.experimental import pallas as pl
from jax.experimental.pallas import tpu as pltpu
from jax.experimental.pallas import tpu_sc as plsc

_NC = 2
_NS = 16
_NW = _NC * _NS
_CHUNK = 2048


def _emb_lookup(xf, table, idm, *, per_w, steps):
    mesh = plsc.VectorSubcoreMesh(core_axis_name="c", subcore_axis_name="s")
    B = xf.shape[0]
    D = table.shape[1]

    @functools.partial(
        pl.kernel,
        mesh=mesh,
        out_type=jax.ShapeDtypeStruct((B, D), jnp.float32),
        scratch_types=[
            pltpu.VMEM((_CHUNK,), jnp.int32),
            pltpu.VMEM((_CHUNK,), jnp.int32),
            pltpu.VMEM((_CHUNK, D), jnp.float32),
            pltpu.SemaphoreType.DMA,
        ],
    )
    def emb_kernel(x_hbm, tab_hbm, map_hbm, out_hbm, xv, mv, rows, sem):
        wid = lax.axis_index("s") * _NC + lax.axis_index("c")

        def body(i, _):
            base = wid * per_w + i * _CHUNK
            pltpu.sync_copy(x_hbm.at[pl.ds(base, _CHUNK)], xv)
            pltpu.async_copy(map_hbm.at[xv], mv, sem).wait()
            pltpu.async_copy(tab_hbm.at[mv], rows, sem).wait()
            pltpu.sync_copy(rows, out_hbm.at[pl.ds(base, _CHUNK)])
            return 0

        lax.fori_loop(0, steps, body, 0)

    return emb_kernel(xf, table, idm)


def kernel(x, table, id_mapper):
    B0, S = x.shape
    V, D = table.shape
    B = B0 * S
    xf = x.reshape(B).astype(jnp.int32)
    idm = id_mapper.astype(jnp.int32)
    per_w = B // _NW
    steps = per_w // _CHUNK
    out = _emb_lookup(xf, table, idm, per_w=per_w, steps=steps)
    return out.reshape(B0, S, D)


# speedup vs baseline: 16.6184x; 16.6184x over previous
import functools

import jax
import jax.numpy as jnp
from jax import lax
from jax.experimental import pallas as pl
from jax.experimental.pallas import tpu as pltpu
from jax.experimental.pallas import tpu_sc as plsc

_NC = 2
_NS = 16
_NW = _NC * _NS
_CHUNK = 2048


def _emb_lookup(xf, table, idm, *, per_w, steps):
    mesh = plsc.VectorSubcoreMesh(core_axis_name="c", subcore_axis_name="s")
    B = xf.shape[0]
    D = table.shape[1]

    @functools.partial(
        pl.kernel,
        mesh=mesh,
        out_type=jax.ShapeDtypeStruct((B, D), jnp.float32),
        compiler_params=pltpu.CompilerParams(use_tc_tiling_on_sc=False),
        scratch_types=[
            pltpu.VMEM((_CHUNK,), jnp.int32),
            pltpu.VMEM((_CHUNK,), jnp.int32),
            pltpu.VMEM((_CHUNK, D), jnp.float32),
            pltpu.SemaphoreType.DMA,
        ],
    )
    def emb_kernel(x_hbm, tab_hbm, map_hbm, out_hbm, xv, mv, rows, sem):
        wid = lax.axis_index("s") * _NC + lax.axis_index("c")

        def body(i, _):
            base = wid * per_w + i * _CHUNK
            pltpu.sync_copy(x_hbm.at[pl.ds(base, _CHUNK)], xv)
            pltpu.async_copy(map_hbm.at[xv], mv, sem).wait()
            pltpu.async_copy(tab_hbm.at[mv], rows, sem).wait()
            pltpu.sync_copy(rows, out_hbm.at[pl.ds(base, _CHUNK)])
            return 0

        lax.fori_loop(0, steps, body, 0)

    return emb_kernel(xf, table, idm)


def kernel(x, table, id_mapper):
    B0, S = x.shape
    V, D = table.shape
    B = B0 * S
    xf = x.reshape(B).astype(jnp.int32)
    idm = id_mapper.astype(jnp.int32)
    per_w = B // _NW
    steps = per_w // _CHUNK
    out = _emb_lookup(xf, table, idm, per_w=per_w, steps=steps)
    return out.reshape(B0, S, D)


# speedup vs baseline: 17.4302x; 1.0488x over previous
import functools

import jax
import jax.numpy as jnp
from jax import lax
from jax.experimental import pallas as pl
from jax.experimental.pallas import tpu as pltpu
from jax.experimental.pallas import tpu_sc as plsc

_NC = 2
_NS = 16
_NW = _NC * _NS
_CHUNK = 1600


def _emb_lookup(xf, table, idm, *, per_w, steps):
    mesh = plsc.VectorSubcoreMesh(core_axis_name="c", subcore_axis_name="s")
    B = xf.shape[0]
    D = table.shape[1]
    C = _CHUNK
    S = steps

    @functools.partial(
        pl.kernel,
        mesh=mesh,
        out_type=jax.ShapeDtypeStruct((B, D), jnp.float32),
        compiler_params=pltpu.CompilerParams(use_tc_tiling_on_sc=False),
        scratch_types=[
            pltpu.VMEM((C,), jnp.int32), pltpu.VMEM((C,), jnp.int32),
            pltpu.VMEM((C,), jnp.int32), pltpu.VMEM((C,), jnp.int32),
            pltpu.VMEM((C, D), jnp.float32), pltpu.VMEM((C, D), jnp.float32),
            pltpu.SemaphoreType.DMA, pltpu.SemaphoreType.DMA,
            pltpu.SemaphoreType.DMA, pltpu.SemaphoreType.DMA,
            pltpu.SemaphoreType.DMA, pltpu.SemaphoreType.DMA,
            pltpu.SemaphoreType.DMA, pltpu.SemaphoreType.DMA,
        ],
    )
    def emb_kernel(x_hbm, tab_hbm, map_hbm, out_hbm,
                   xv0, xv1, mv0, mv1, rw0, rw1,
                   sx0, sx1, sm0, sm1, st0, st1, so0, so1):
        wid = lax.axis_index("s") * _NC + lax.axis_index("c")
        base0 = wid * per_w
        xv = (xv0, xv1)
        mv = (mv0, mv1)
        rw = (rw0, rw1)
        sx = (sx0, sx1)
        sm = (sm0, sm1)
        st = (st0, st1)
        so = (so0, so1)

        def start_x(c, b):
            pltpu.async_copy(x_hbm.at[pl.ds(base0 + c * C, C)], xv[b], sx[b])

        def wait_x(b):
            pltpu.make_async_copy(x_hbm.at[pl.ds(base0, C)], xv[b], sx[b]).wait()

        def start_map(b):
            pltpu.async_copy(map_hbm.at[xv[b]], mv[b], sm[b])

        def wait_map(b):
            pltpu.make_async_copy(map_hbm.at[xv[b]], mv[b], sm[b]).wait()

        def start_tab(b):
            pltpu.async_copy(tab_hbm.at[mv[b]], rw[b], st[b])

        def wait_tab(b):
            pltpu.make_async_copy(tab_hbm.at[mv[b]], rw[b], st[b]).wait()

        def start_out(c, b):
            pltpu.async_copy(rw[b], out_hbm.at[pl.ds(base0 + c * C, C)], so[b])

        def wait_out(b):
            pltpu.make_async_copy(rw[b], out_hbm.at[pl.ds(base0, C)], so[b]).wait()

        start_x(0, 0)
        start_x(1, 1)
        wait_x(0)
        start_map(0)

        def stage(t, b):
            nb = 1 - b

            @pl.when(t < S)
            def _():
                wait_map(b)

            @pl.when(t >= 2)
            def _():
                wait_out(b)

            @pl.when(t < S)
            def _():
                start_tab(b)

            @pl.when(jnp.logical_and(t >= 1, t - 1 < S))
            def _():
                wait_tab(nb)
                start_out(t - 1, nb)

            @pl.when(t + 1 < S)
            def _():
                wait_x(nb)
                start_map(nb)

            @pl.when(t + 2 < S)
            def _():
                start_x(t + 2, b)

        def body(i, _):
            t0 = i * 2
            stage(t0, 0)
            stage(t0 + 1, 1)
            return 0

        lax.fori_loop(0, (S + 2) // 2, body, 0)

    return emb_kernel(xf, table, idm)


def kernel(x, table, id_mapper):
    B0, Sq = x.shape
    V, D = table.shape
    B = B0 * Sq
    xf = x.reshape(B).astype(jnp.int32)
    idm = id_mapper.astype(jnp.int32)
    per_w = B // _NW
    steps = per_w // _CHUNK
    out = _emb_lookup(xf, table, idm, per_w=per_w, steps=steps)
    return out.reshape(B0, Sq, D)
